# baseline (speedup 1.0000x reference)
import functools

import jax
import jax.numpy as jnp
from jax import lax
from jax.experimental import pallas as pl
from jax.experimental.pallas import tpu as pltpu
from jax.experimental.pallas import tpu_sc as plsc

_NU, _NI, _NB, _D = 50000, 40000, 20000, 64
_L = 2

_TILES = 16
_CORES = 2
_W = 256
_VLEN = 16
_RANGE = 10752
_RPT = _RANGE // _TILES
_ZC = 224
_EDGE_QUANT = _TILES * _W

_N1 = _NU + _NI
_N2 = _NU + _NB
_NACC1 = 9 * _RANGE
_NACC2 = 7 * _RANGE
_NACC3 = 2 * _RANGE


def _round_up(x, m):
    return (x + m - 1) // m * m


def _mesh():
    return plsc.VectorSubcoreMesh(
        core_axis_name="c", subcore_axis_name="s",
        num_cores=_CORES, num_subcores=_TILES,
    )


def _make_segsum(nacc, e2p, ntab, ones_mode):
    nwin = e2p // _EDGE_QUANT
    nranges = nacc // _RANGE
    passes = -(-nranges // _CORES)
    dummy = ntab - 1

    @functools.partial(
        pl.kernel,
        out_type=jax.ShapeDtypeStruct((nacc, 128), jnp.float32),
        mesh=_mesh(),
        scratch_types=[
            pltpu.VMEM((_W,), jnp.int32),
            pltpu.VMEM((_W,), jnp.int32),
            pltpu.VMEM((_W, 128), jnp.float32),
            pltpu.VMEM_SHARED((_RANGE, 128), jnp.float32),
        ],
    )
    def segsum(table, src, dst, out, idx_s, idx_d, rows, accum):
        c = lax.axis_index("c")
        s = lax.axis_index("s")
        for p in range(passes):
            r = _CORES * p + c
            live = r < nranges

            @pl.when(live)
            def _():
                base = r * _RANGE
                @pl.loop(0, _W)
                def _(i):
                    @pl.loop(0, 128, step=_VLEN)
                    def _(j):
                        rows[pl.ds(i, 1), pl.ds(j, _VLEN)] = jnp.zeros(
                            (1, _VLEN), jnp.float32)

                @pl.loop(0, _RPT // _ZC)
                def _(z):
                    pltpu.sync_copy(
                        rows.at[pl.ds(0, _ZC)],
                        accum.at[pl.ds(s * _RPT + z * _ZC, _ZC)],
                    )
                plsc.subcore_barrier()

                @pl.loop(0, nwin)
                def _(w):
                    wbase = (s * nwin + w) * _W
                    pltpu.sync_copy(src.at[pl.ds(wbase, _W)], idx_s)
                    pltpu.sync_copy(dst.at[pl.ds(wbase, _W)], idx_d)

                    @pl.loop(0, _W, step=_VLEN)
                    def _(i):
                        sl = pl.ds(i, _VLEN)
                        lv = idx_d[sl] - base
                        inr = (lv >= 0) & (lv < _RANGE)
                        if ones_mode:
                            sv = jnp.where(inr, 1, 0)
                        else:
                            sv = jnp.where(inr, idx_s[sl], dummy)
                        idx_s[sl] = sv
                        idx_d[sl] = jnp.where(inr, lv, 0)

                    pltpu.sync_copy(table.at[idx_s], rows)
                    pltpu.sync_copy(rows, accum.at[idx_d], add=True)

                plsc.subcore_barrier()
                pltpu.sync_copy(
                    accum.at[pl.ds(s * _RPT, _RPT)],
                    out.at[pl.ds(base + s * _RPT, _RPT)],
                )
                plsc.subcore_barrier()

    return segsum


_BN = 512


def _dis(deg_block):
    return 1.0 / (jnp.sqrt(deg_block[:, 0:1]) + 1e-8)


def _scale_in(f0, deg):
    nacc = f0.shape[0]

    def body(f_ref, deg_ref, g_ref):
        g_ref[...] = f_ref[...] * _dis(deg_ref)

    return pl.pallas_call(
        body,
        grid=(nacc // _BN,),
        in_specs=[
            pl.BlockSpec((_BN, 128), lambda i: (i, 0)),
            pl.BlockSpec((_BN, 128), lambda i: (i, 0)),
        ],
        out_specs=pl.BlockSpec((_BN, 128), lambda i: (i, 0)),
        out_shape=jax.ShapeDtypeStruct((nacc, 128), jnp.float32),
    )(f0, deg)


def _post_layer(ssum, deg, acc_in, layer, want_g):
    nacc = ssum.shape[0]

    def body(s_ref, deg_ref, acc_ref, *out_refs):
        dis = _dis(deg_ref)
        f = s_ref[...] * dis / float(layer + 2)
        nrm = jnp.maximum(
            jnp.sqrt(jnp.sum(f * f, axis=1, keepdims=True)), 1e-12
        )
        out_refs[0][...] = acc_ref[...] + f / nrm
        if want_g:
            out_refs[1][...] = f * dis

    nout = 2 if want_g else 1
    return pl.pallas_call(
        body,
        grid=(nacc // _BN,),
        in_specs=[
            pl.BlockSpec((_BN, 128), lambda i: (i, 0)),
            pl.BlockSpec((_BN, 128), lambda i: (i, 0)),
            pl.BlockSpec((_BN, 128), lambda i: (i, 0)),
        ],
        out_specs=[pl.BlockSpec((_BN, 128), lambda i: (i, 0))] * nout,
        out_shape=[jax.ShapeDtypeStruct((nacc, 128), jnp.float32)] * nout,
    )(ssum, deg, acc_in)


_BNC = 400


def _combine_users(acc_ui, acc_ub):
    def body(a_ref, b_ref, o_ref):
        o_ref[...] = a_ref[:, 0:64] + b_ref[:, 0:64]

    return pl.pallas_call(
        body,
        grid=(_NU // _BNC,),
        in_specs=[
            pl.BlockSpec((_BNC, 128), lambda i: (i, 0)),
            pl.BlockSpec((_BNC, 128), lambda i: (i, 0)),
        ],
        out_specs=pl.BlockSpec((_BNC, 64), lambda i: (i, 0)),
        out_shape=jax.ShapeDtypeStruct((_NU, 64), jnp.float32),
    )(acc_ui, acc_ub)


def _combine_bundles(sb, deg_b, acc_ub):
    off = _NU // _BNC

    def body(s_ref, deg_ref, a_ref, o_ref):
        val = 1.0 / (deg_ref[:, 0:1] + 1e-8)
        o_ref[...] = s_ref[:, 0:64] * val + a_ref[:, 0:64]

    return pl.pallas_call(
        body,
        grid=(_NB // _BNC,),
        in_specs=[
            pl.BlockSpec((_BNC, 128), lambda i: (i, 0)),
            pl.BlockSpec((_BNC, 128), lambda i: (i, 0)),
            pl.BlockSpec((_BNC, 128), lambda i: (i + off, 0)),
        ],
        out_specs=pl.BlockSpec((_BNC, 64), lambda i: (i, 0)),
        out_shape=jax.ShapeDtypeStruct((_NB, 64), jnp.float32),
    )(sb, deg_b, acc_ub)


def _padded_bipartite(rows, cols, n_a, dummy_src, dummy_dst, e2p):
    rows = rows.astype(jnp.int32)
    shifted = (cols + n_a).astype(jnp.int32)
    src = jnp.concatenate([shifted, rows])
    dst = jnp.concatenate([rows, shifted])
    pad = e2p - src.shape[0]
    src = jnp.concatenate([src, jnp.full((pad,), dummy_src, jnp.int32)])
    dst = jnp.concatenate([dst, jnp.full((pad,), dummy_dst, jnp.int32)])
    return src, dst


def _pad_nodes(feat_a, feat_b, nacc):
    n = feat_a.shape[0] + feat_b.shape[0]
    stacked = jnp.concatenate(
        [feat_a, feat_b, jnp.zeros((nacc - n, _D), jnp.float32)], axis=0
    )
    return jnp.concatenate(
        [stacked, jnp.zeros((nacc, 128 - _D), jnp.float32)], axis=1
    )


def _propagate(seg, deg, src, dst, f0):
    g = _scale_in(f0, deg)
    acc = f0
    for layer in range(_L):
        ssum = seg(g, src, dst)
        outs = _post_layer(ssum, deg, acc, layer, want_g=layer < _L - 1)
        acc = outs[0]
        if layer < _L - 1:
            g = outs[1]
    return acc


def kernel(users_feature, items_feature, bundles_feature,
           ui_rows, ui_cols, ub_rows, ub_cols, bi_rows, bi_cols):
    e1p = _round_up(2 * ui_rows.shape[0], _EDGE_QUANT)
    e2p = _round_up(2 * ub_rows.shape[0], _EDGE_QUANT)
    e3p = _round_up(bi_rows.shape[0], _EDGE_QUANT)

    src1, dst1 = _padded_bipartite(ui_rows, ui_cols, _NU, _NACC1 - 1,
                                   _NACC1, e1p)
    src2, dst2 = _padded_bipartite(ub_rows, ub_cols, _NU, _NACC2 - 1,
                                   _NACC2, e2p)
    pad3 = e3p - bi_rows.shape[0]
    src3 = jnp.concatenate(
        [(bi_cols + _NU).astype(jnp.int32),
         jnp.full((pad3,), _NACC1 - 1, jnp.int32)])
    dst3 = jnp.concatenate(
        [bi_rows.astype(jnp.int32), jnp.full((pad3,), _NACC3, jnp.int32)])

    table01 = jnp.concatenate(
        [jnp.zeros((1, 128), jnp.float32), jnp.ones((1, 128), jnp.float32)])
    deg1 = _make_segsum(_NACC1, e1p, 2, True)(table01, src1, dst1)
    deg2 = _make_segsum(_NACC2, e2p, 2, True)(table01, src2, dst2)
    deg3 = _make_segsum(_NACC3, e3p, 2, True)(table01, src3, dst3)

    f0_ui = _pad_nodes(users_feature, items_feature, _NACC1)
    f0_ub = _pad_nodes(users_feature, bundles_feature, _NACC2)

    seg1 = _make_segsum(_NACC1, e1p, _NACC1, False)
    seg2 = _make_segsum(_NACC2, e2p, _NACC2, False)
    acc_ui = _propagate(seg1, deg1, src1, dst1, f0_ui)
    acc_ub = _propagate(seg2, deg2, src2, dst2, f0_ub)

    sb = _make_segsum(_NACC3, e3p, _NACC1, False)(acc_ui, src3, dst3)

    users_rep = _combine_users(acc_ui, acc_ub)
    bundles_rep = _combine_bundles(sb, deg3, acc_ub)
    return jnp.concatenate([users_rep, bundles_rep], axis=0)


# speedup vs baseline: 1.0955x; 1.0955x over previous
import functools

import jax
import jax.numpy as jnp
from jax import lax
from jax.experimental import pallas as pl
from jax.experimental.pallas import tpu as pltpu
from jax.experimental.pallas import tpu_sc as plsc

_NU, _NI, _NB, _D = 50000, 40000, 20000, 64
_L = 2

_TILES = 16
_CORES = 2
_W = 256
_VLEN = 16
_RANGE = 8192
_TRASH = 8
_RPT = _RANGE // _TILES
_ZC = 128
_EDGE_QUANT = 8 * 8192

_N1 = _NU + _NI
_N2 = _NU + _NB
_NACC1 = 11 * _RANGE
_NACC2 = 9 * _RANGE
_NACC3 = 3 * _RANGE


def _round_up(x, m):
    return (x + m - 1) // m * m


def _mesh():
    return plsc.VectorSubcoreMesh(
        core_axis_name="c", subcore_axis_name="s",
        num_cores=_CORES, num_subcores=_TILES,
    )


def _premask(src, dst, nranges, nacc, dummy_src):
    e2p = src.shape[0]
    eb = e2p // 8192

    def body(s_ref, d_ref, ms_ref, md_ref):
        base = pl.program_id(0) * _RANGE
        sv = s_ref[...]
        dv = d_ref[...]
        inr = (dv >= base) & (dv < base + _RANGE)
        trash = _RANGE + (
            jax.lax.broadcasted_iota(jnp.int32, (8, 8192), 1) & (_TRASH - 1))
        ms_ref[0] = jnp.where(inr, sv, dummy_src)
        md_ref[0] = jnp.where(inr, dv - base, trash)

    ms, md = pl.pallas_call(
        body,
        grid=(nranges, eb // 8),
        in_specs=[
            pl.BlockSpec((8, 8192), lambda r, j: (j, 0)),
            pl.BlockSpec((8, 8192), lambda r, j: (j, 0)),
        ],
        out_specs=[
            pl.BlockSpec((1, 8, 8192), lambda r, j: (r, j, 0)),
            pl.BlockSpec((1, 8, 8192), lambda r, j: (r, j, 0)),
        ],
        out_shape=[
            jax.ShapeDtypeStruct((nranges, eb, 8192), jnp.int32),
            jax.ShapeDtypeStruct((nranges, eb, 8192), jnp.int32),
        ],
    )(src.reshape(eb, 8192), dst.reshape(eb, 8192))
    return ms.reshape(-1), md.reshape(-1)


def _fill(buf, value):
    row = jnp.full((1, _VLEN), value, jnp.float32)

    @pl.loop(0, buf.shape[0])
    def _(i):
        @pl.loop(0, 128, step=_VLEN)
        def _(j):
            buf[pl.ds(i, 1), pl.ds(j, _VLEN)] = row


def _make_segsum(nacc, e2p, ones_mode):
    nwin = e2p // (_TILES * _W)
    nranges = nacc // _RANGE
    passes = -(-nranges // _CORES)

    @functools.partial(
        pl.kernel,
        out_type=jax.ShapeDtypeStruct((nacc, 128), jnp.float32),
        mesh=_mesh(),
        scratch_types=[
            pltpu.VMEM((_W,), jnp.int32),
            pltpu.VMEM((_W,), jnp.int32),
            pltpu.VMEM((_W, 128), jnp.float32),
            pltpu.VMEM((_ZC, 128), jnp.float32),
            pltpu.VMEM_SHARED((_RANGE + _TRASH, 128), jnp.float32),
        ],
    )
    def segsum(table, msrc, mdst, out, idx_s, idx_d, rows, zbuf, accum):
        c = lax.axis_index("c")
        s = lax.axis_index("s")
        _fill(zbuf, 0.0)
        if ones_mode:
            _fill(rows, 1.0)
        for p in range(passes):
            r = _CORES * p + c
            live = r < nranges

            @pl.when(live)
            def _():
                @pl.loop(0, _RPT // _ZC)
                def _(z):
                    pltpu.sync_copy(
                        zbuf, accum.at[pl.ds(s * _RPT + z * _ZC, _ZC)])

                @pl.when(s == 0)
                def _():
                    pltpu.sync_copy(zbuf.at[pl.ds(0, _TRASH)],
                                    accum.at[pl.ds(_RANGE, _TRASH)])
                plsc.subcore_barrier()

                rbase = r * e2p

                @pl.loop(0, nwin)
                def _(w):
                    wbase = rbase + (s * nwin + w) * _W
                    pltpu.sync_copy(mdst.at[pl.ds(wbase, _W)], idx_d)
                    if not ones_mode:
                        pltpu.sync_copy(msrc.at[pl.ds(wbase, _W)], idx_s)
                        pltpu.sync_copy(table.at[idx_s], rows)
                    pltpu.sync_copy(rows, accum.at[idx_d], add=True)

                plsc.subcore_barrier()
                pltpu.sync_copy(
                    accum.at[pl.ds(s * _RPT, _RPT)],
                    out.at[pl.ds(r * _RANGE + s * _RPT, _RPT)],
                )
                plsc.subcore_barrier()

    return segsum


_BN = 512


def _dis(deg_block):
    return 1.0 / (jnp.sqrt(deg_block[:, 0:1]) + 1e-8)


def _scale_in(f0, deg):
    nacc = f0.shape[0]

    def body(f_ref, deg_ref, g_ref):
        g_ref[...] = f_ref[...] * _dis(deg_ref)

    return pl.pallas_call(
        body,
        grid=(nacc // _BN,),
        in_specs=[
            pl.BlockSpec((_BN, 128), lambda i: (i, 0)),
            pl.BlockSpec((_BN, 128), lambda i: (i, 0)),
        ],
        out_specs=pl.BlockSpec((_BN, 128), lambda i: (i, 0)),
        out_shape=jax.ShapeDtypeStruct((nacc, 128), jnp.float32),
    )(f0, deg)


def _post_layer(ssum, deg, acc_in, layer, want_g):
    nacc = ssum.shape[0]

    def body(s_ref, deg_ref, acc_ref, *out_refs):
        dis = _dis(deg_ref)
        f = s_ref[...] * dis / float(layer + 2)
        nrm = jnp.maximum(
            jnp.sqrt(jnp.sum(f * f, axis=1, keepdims=True)), 1e-12
        )
        out_refs[0][...] = acc_ref[...] + f / nrm
        if want_g:
            out_refs[1][...] = f * dis

    nout = 2 if want_g else 1
    return pl.pallas_call(
        body,
        grid=(nacc // _BN,),
        in_specs=[
            pl.BlockSpec((_BN, 128), lambda i: (i, 0)),
            pl.BlockSpec((_BN, 128), lambda i: (i, 0)),
            pl.BlockSpec((_BN, 128), lambda i: (i, 0)),
        ],
        out_specs=[pl.BlockSpec((_BN, 128), lambda i: (i, 0))] * nout,
        out_shape=[jax.ShapeDtypeStruct((nacc, 128), jnp.float32)] * nout,
    )(ssum, deg, acc_in)


_BNC = 400


def _combine_users(acc_ui, acc_ub):
    def body(a_ref, b_ref, o_ref):
        o_ref[...] = a_ref[:, 0:64] + b_ref[:, 0:64]

    return pl.pallas_call(
        body,
        grid=(_NU // _BNC,),
        in_specs=[
            pl.BlockSpec((_BNC, 128), lambda i: (i, 0)),
            pl.BlockSpec((_BNC, 128), lambda i: (i, 0)),
        ],
        out_specs=pl.BlockSpec((_BNC, 64), lambda i: (i, 0)),
        out_shape=jax.ShapeDtypeStruct((_NU, 64), jnp.float32),
    )(acc_ui, acc_ub)


def _combine_bundles(sb, deg_b, acc_ub):
    off = _NU // _BNC

    def body(s_ref, deg_ref, a_ref, o_ref):
        val = 1.0 / (deg_ref[:, 0:1] + 1e-8)
        o_ref[...] = s_ref[:, 0:64] * val + a_ref[:, 0:64]

    return pl.pallas_call(
        body,
        grid=(_NB // _BNC,),
        in_specs=[
            pl.BlockSpec((_BNC, 128), lambda i: (i, 0)),
            pl.BlockSpec((_BNC, 128), lambda i: (i, 0)),
            pl.BlockSpec((_BNC, 128), lambda i: (i + off, 0)),
        ],
        out_specs=pl.BlockSpec((_BNC, 64), lambda i: (i, 0)),
        out_shape=jax.ShapeDtypeStruct((_NB, 64), jnp.float32),
    )(sb, deg_b, acc_ub)


def _padded_bipartite(rows, cols, n_a, dummy_src, dummy_dst, e2p):
    rows = rows.astype(jnp.int32)
    shifted = (cols + n_a).astype(jnp.int32)
    src = jnp.concatenate([shifted, rows])
    dst = jnp.concatenate([rows, shifted])
    pad = e2p - src.shape[0]
    src = jnp.concatenate([src, jnp.full((pad,), dummy_src, jnp.int32)])
    dst = jnp.concatenate([dst, jnp.full((pad,), dummy_dst, jnp.int32)])
    return src, dst


def _pad_nodes(feat_a, feat_b, nacc):
    n = feat_a.shape[0] + feat_b.shape[0]
    stacked = jnp.concatenate(
        [feat_a, feat_b, jnp.zeros((nacc - n, _D), jnp.float32)], axis=0
    )
    return jnp.concatenate(
        [stacked, jnp.zeros((nacc, 128 - _D), jnp.float32)], axis=1
    )


def _propagate(seg, deg, msrc, mdst, f0):
    g = _scale_in(f0, deg)
    acc = f0
    for layer in range(_L):
        ssum = seg(g, msrc, mdst)
        outs = _post_layer(ssum, deg, acc, layer, want_g=layer < _L - 1)
        acc = outs[0]
        if layer < _L - 1:
            g = outs[1]
    return acc


def kernel(users_feature, items_feature, bundles_feature,
           ui_rows, ui_cols, ub_rows, ub_cols, bi_rows, bi_cols):
    e1p = _round_up(2 * ui_rows.shape[0], _EDGE_QUANT)
    e2p = _round_up(2 * ub_rows.shape[0], _EDGE_QUANT)
    e3p = _round_up(bi_rows.shape[0], _EDGE_QUANT)

    src1, dst1 = _padded_bipartite(ui_rows, ui_cols, _NU, _NACC1 - 1,
                                   _NACC1, e1p)
    src2, dst2 = _padded_bipartite(ub_rows, ub_cols, _NU, _NACC2 - 1,
                                   _NACC2, e2p)
    pad3 = e3p - bi_rows.shape[0]
    src3 = jnp.concatenate(
        [(bi_cols + _NU).astype(jnp.int32),
         jnp.full((pad3,), _NACC1 - 1, jnp.int32)])
    dst3 = jnp.concatenate(
        [bi_rows.astype(jnp.int32), jnp.full((pad3,), _NACC3, jnp.int32)])

    ms1, md1 = _premask(src1, dst1, _NACC1 // _RANGE, _NACC1, _NACC1 - 1)
    ms2, md2 = _premask(src2, dst2, _NACC2 // _RANGE, _NACC2, _NACC2 - 1)
    ms3, md3 = _premask(src3, dst3, _NACC3 // _RANGE, _NACC3, _NACC1 - 1)

    table0 = jnp.zeros((2, 128), jnp.float32)
    deg1 = _make_segsum(_NACC1, e1p, True)(table0, md1, md1)
    deg2 = _make_segsum(_NACC2, e2p, True)(table0, md2, md2)
    deg3 = _make_segsum(_NACC3, e3p, True)(table0, md3, md3)

    f0_ui = _pad_nodes(users_feature, items_feature, _NACC1)
    f0_ub = _pad_nodes(users_feature, bundles_feature, _NACC2)

    seg1 = _make_segsum(_NACC1, e1p, False)
    seg2 = _make_segsum(_NACC2, e2p, False)
    acc_ui = _propagate(seg1, deg1, ms1, md1, f0_ui)
    acc_ub = _propagate(seg2, deg2, ms2, md2, f0_ub)

    sb = _make_segsum(_NACC3, e3p, False)(acc_ui, ms3, md3)

    users_rep = _combine_users(acc_ui, acc_ub)
    bundles_rep = _combine_bundles(sb, deg3, acc_ub)
    return jnp.concatenate([users_rep, bundles_rep], axis=0)
